# Initial kernel scaffold; baseline (speedup 1.0000x reference)
import jax
import jax.numpy as jnp
from jax.experimental import pallas as pl


def kernel(x, W0, b0, W1, b1, We, be, Wg, bg):
    raise NotImplementedError("write your pallas kernel here")


# speedup vs baseline: 12.1501x; 12.1501x over previous
import functools

import jax
import jax.numpy as jnp
from jax.experimental import pallas as pl
from jax.experimental.pallas import tpu as pltpu

B = 8
C = 3
H = 352
N = 484
NPAD = 512
D = 768
K = 9
NEG = -3e38


def _rb(v):
    return v.astype(jnp.bfloat16).astype(jnp.float32)


def _conv_mul_kernel(w0_ref, b0_ref, w1_ref, b1_ref, xp_ref, amp_ref, prompt_ref,
                     h_scratch):
    xp = xp_ref[0]
    xpb = _rb(xp)
    h_scratch[...] = jnp.zeros_like(h_scratch)
    for co in range(C):
        acc = jnp.zeros((H, H), jnp.float32)
        for ci in range(C):
            for dh in range(3):
                for dw in range(3):
                    w = _rb(w0_ref[((co * C + ci) * 3 + dh) * 3 + dw])
                    acc = acc + w * xpb[ci, dh:dh + H, dw:dw + H]
        h_scratch[co, 1:H + 1, 1:H + 1] = jnp.maximum(acc + b0_ref[co], 0.0)
    h = _rb(h_scratch[...])
    for co in range(C):
        acc = jnp.zeros((H, H), jnp.float32)
        for ci in range(C):
            for dh in range(3):
                for dw in range(3):
                    w = _rb(w1_ref[((co * C + ci) * 3 + dh) * 3 + dw])
                    acc = acc + w * h[ci, dh:dh + H, dw:dw + H]
        pr = acc + b1_ref[co]
        prompt_ref[0, co] = pr
        amp_ref[0, co] = pr * xp[co, 1:H + 1, 1:H + 1]


def _embed_dist_kernel(p_ref, we_ref, be_ref, feat_ref, dist_ref):
    pb = p_ref[0].astype(jnp.bfloat16)
    feat = jnp.dot(pb, we_ref[...], preferred_element_type=jnp.float32)
    feat = feat + be_ref[...]
    feat_ref[0] = feat
    sq = jnp.sum(feat * feat, axis=1, keepdims=True)
    fb = feat.astype(jnp.bfloat16)
    gram = jax.lax.dot_general(fb, fb, (((1,), (1,)), ((), ())),
                               preferred_element_type=jnp.float32)
    dist = sq + sq.T - 2.0 * gram
    col = jax.lax.broadcasted_iota(jnp.int32, (NPAD, NPAD), 1)
    dist_ref[0] = jnp.where(col < N, dist, jnp.inf)


def _topk_gnn_kernel(feat_ref, dist_ref, wgt_ref, wgb_ref, bg_ref, out_ref):
    feat = feat_ref[0]
    dist = dist_ref[0]
    col = jax.lax.broadcasted_iota(jnp.int32, (NPAD, NPAD), 1)
    maxrel = jnp.full((NPAD, D), NEG, jnp.float32)
    fb = feat.astype(jnp.bfloat16)
    for _ in range(K):
        rowmin = jnp.min(dist, axis=1, keepdims=True)
        cand = jnp.where(dist == rowmin, col, NPAD)
        sel = jnp.min(cand, axis=1, keepdims=True)
        onehot = (col == sel)
        nb = jnp.dot(onehot.astype(jnp.bfloat16), fb,
                     preferred_element_type=jnp.float32)
        maxrel = jnp.maximum(maxrel, nb)
        dist = jnp.where(onehot, jnp.inf, dist)
    maxrel = maxrel - feat
    h = jnp.dot(fb, wgt_ref[...], preferred_element_type=jnp.float32)
    h = h + jnp.dot(maxrel.astype(jnp.bfloat16), wgb_ref[...],
                    preferred_element_type=jnp.float32)
    h = h + bg_ref[...]
    out_ref[0] = feat + jnp.maximum(h, 0.0)


def kernel(x, W0, b0, W1, b1, We, be, Wg, bg):
    xp = jnp.pad(x, ((0, 0), (0, 0), (1, 1), (1, 1)))
    w0f = W0.reshape(-1)
    w1f = W1.reshape(-1)

    amp_src, prompt1 = pl.pallas_call(
        _conv_mul_kernel,
        grid=(B,),
        in_specs=[
            pl.BlockSpec(memory_space=pltpu.SMEM),
            pl.BlockSpec(memory_space=pltpu.SMEM),
            pl.BlockSpec(memory_space=pltpu.SMEM),
            pl.BlockSpec(memory_space=pltpu.SMEM),
            pl.BlockSpec((1, C, H + 2, H + 2), lambda i: (i, 0, 0, 0)),
        ],
        out_specs=[
            pl.BlockSpec((1, C, H, H), lambda i: (i, 0, 0, 0)),
            pl.BlockSpec((1, C, H, H), lambda i: (i, 0, 0, 0)),
        ],
        out_shape=[
            jax.ShapeDtypeStruct((B, C, H, H), jnp.float32),
            jax.ShapeDtypeStruct((B, C, H, H), jnp.float32),
        ],
        scratch_shapes=[pltpu.VMEM((C, H + 2, H + 2), jnp.float32)],
    )(w0f, b0, w1f, b1, xp)

    p = prompt1.reshape(B, C, 22, 16, 22, 16)
    p = p.transpose(0, 2, 4, 1, 3, 5).reshape(B, N, C * 16 * 16)
    p = jnp.pad(p, ((0, 0), (0, NPAD - N), (0, 0)))

    feat, dist = pl.pallas_call(
        _embed_dist_kernel,
        grid=(B,),
        in_specs=[
            pl.BlockSpec((1, NPAD, D), lambda i: (i, 0, 0)),
            pl.BlockSpec((D, D), lambda i: (0, 0)),
            pl.BlockSpec((1, D), lambda i: (0, 0)),
        ],
        out_specs=[
            pl.BlockSpec((1, NPAD, D), lambda i: (i, 0, 0)),
            pl.BlockSpec((1, NPAD, NPAD), lambda i: (i, 0, 0)),
        ],
        out_shape=[
            jax.ShapeDtypeStruct((B, NPAD, D), jnp.float32),
            jax.ShapeDtypeStruct((B, NPAD, NPAD), jnp.float32),
        ],
    )(p, We.astype(jnp.bfloat16), be.reshape(1, D))

    out = pl.pallas_call(
        _topk_gnn_kernel,
        grid=(B,),
        in_specs=[
            pl.BlockSpec((1, NPAD, D), lambda i: (i, 0, 0)),
            pl.BlockSpec((1, NPAD, NPAD), lambda i: (i, 0, 0)),
            pl.BlockSpec((D, D), lambda i: (0, 0)),
            pl.BlockSpec((D, D), lambda i: (0, 0)),
            pl.BlockSpec((1, D), lambda i: (0, 0)),
        ],
        out_specs=pl.BlockSpec((1, NPAD, D), lambda i: (i, 0, 0)),
        out_shape=jax.ShapeDtypeStruct((B, NPAD, D), jnp.float32),
    )(feat, dist, Wg[:D].astype(jnp.bfloat16), Wg[D:].astype(jnp.bfloat16),
      bg.reshape(1, D))

    amp_low = out[:, :N, :]
    return (amp_src, amp_low)


# speedup vs baseline: 18.6449x; 1.5345x over previous
import functools

import jax
import jax.numpy as jnp
from jax.experimental import pallas as pl
from jax.experimental.pallas import tpu as pltpu

B = 8
C = 3
H = 352
N = 484
NPAD = 512
D = 768
K = 9
NEG = -3e38


def _rb(v):
    return v.astype(jnp.bfloat16).astype(jnp.float32)


def _conv_patch_kernel(w0_ref, b0_ref, w1_ref, b1_ref, x_ref, amp_ref, p_ref,
                       xs, hs):
    @pl.when(pl.program_id(0) == 0)
    def _init():
        xs[...] = jnp.zeros_like(xs)
        hs[...] = jnp.zeros_like(hs)

    xs[:, 1:H + 1, 1:H + 1] = x_ref[0]
    xp = xs[...]
    xpb = _rb(xp)
    for co in range(C):
        acc = jnp.zeros((H, H), jnp.float32)
        for ci in range(C):
            for dh in range(3):
                for dw in range(3):
                    w = _rb(w0_ref[((co * C + ci) * 3 + dh) * 3 + dw])
                    acc = acc + w * xpb[ci, dh:dh + H, dw:dw + H]
        hs[co, 1:H + 1, 1:H + 1] = jnp.maximum(acc + b0_ref[co], 0.0)
    h = _rb(hs[...])
    prs = []
    for co in range(C):
        acc = jnp.zeros((H, H), jnp.float32)
        for ci in range(C):
            for dh in range(3):
                for dw in range(3):
                    w = _rb(w1_ref[((co * C + ci) * 3 + dh) * 3 + dw])
                    acc = acc + w * h[ci, dh:dh + H, dw:dw + H]
        pr = acc + b1_ref[co]
        amp_ref[0, co] = pr * xp[co, 1:H + 1, 1:H + 1]
        prs.append(pr)
    v = jnp.stack(prs)
    v = v.reshape(C, 22, 16, 22, 16)
    v = jnp.transpose(v, (1, 3, 0, 2, 4))
    v = v.reshape(N, D)
    p_ref[0] = jnp.concatenate([v, jnp.zeros((NPAD - N, D), jnp.float32)], axis=0)


def _vig_kernel(p_ref, we_ref, be_ref, wgt_ref, wgb_ref, bg_ref, out_ref):
    pb = p_ref[0].astype(jnp.bfloat16)
    feat = jnp.dot(pb, we_ref[...], preferred_element_type=jnp.float32)
    feat = feat + be_ref[...]
    sq = jnp.sum(feat * feat, axis=1, keepdims=True)
    fb = feat.astype(jnp.bfloat16)
    gram = jax.lax.dot_general(fb, fb, (((1,), (1,)), ((), ())),
                               preferred_element_type=jnp.float32)
    dist = sq + sq.T - 2.0 * gram
    col = jax.lax.broadcasted_iota(jnp.int32, (NPAD, NPAD), 1)
    dist = jnp.where(col < N, dist, jnp.inf)
    maxrel = jnp.full((NPAD, D), NEG, jnp.float32)
    for _ in range(K):
        rowmin = jnp.min(dist, axis=1, keepdims=True)
        cand = jnp.where(dist == rowmin, col, NPAD)
        sel = jnp.min(cand, axis=1, keepdims=True)
        onehot = (col == sel)
        nb = jnp.dot(onehot.astype(jnp.bfloat16), fb,
                     preferred_element_type=jnp.float32)
        maxrel = jnp.maximum(maxrel, nb)
        dist = jnp.where(onehot, jnp.inf, dist)
    maxrel = maxrel - feat
    hh = jnp.dot(fb, wgt_ref[...], preferred_element_type=jnp.float32)
    hh = hh + jnp.dot(maxrel.astype(jnp.bfloat16), wgb_ref[...],
                      preferred_element_type=jnp.float32)
    hh = hh + bg_ref[...]
    out_ref[0] = feat + jnp.maximum(hh, 0.0)


def kernel(x, W0, b0, W1, b1, We, be, Wg, bg):
    w0f = W0.reshape(-1)
    w1f = W1.reshape(-1)

    amp_src, p = pl.pallas_call(
        _conv_patch_kernel,
        grid=(B,),
        in_specs=[
            pl.BlockSpec(memory_space=pltpu.SMEM),
            pl.BlockSpec(memory_space=pltpu.SMEM),
            pl.BlockSpec(memory_space=pltpu.SMEM),
            pl.BlockSpec(memory_space=pltpu.SMEM),
            pl.BlockSpec((1, C, H, H), lambda i: (i, 0, 0, 0)),
        ],
        out_specs=[
            pl.BlockSpec((1, C, H, H), lambda i: (i, 0, 0, 0)),
            pl.BlockSpec((1, NPAD, D), lambda i: (i, 0, 0)),
        ],
        out_shape=[
            jax.ShapeDtypeStruct((B, C, H, H), jnp.float32),
            jax.ShapeDtypeStruct((B, NPAD, D), jnp.float32),
        ],
        scratch_shapes=[
            pltpu.VMEM((C, H + 2, H + 2), jnp.float32),
            pltpu.VMEM((C, H + 2, H + 2), jnp.float32),
        ],
    )(w0f, b0, w1f, b1, x)

    out = pl.pallas_call(
        _vig_kernel,
        grid=(B,),
        in_specs=[
            pl.BlockSpec((1, NPAD, D), lambda i: (i, 0, 0)),
            pl.BlockSpec((D, D), lambda i: (0, 0)),
            pl.BlockSpec((1, D), lambda i: (0, 0)),
            pl.BlockSpec((D, D), lambda i: (0, 0)),
            pl.BlockSpec((D, D), lambda i: (0, 0)),
            pl.BlockSpec((1, D), lambda i: (0, 0)),
        ],
        out_specs=pl.BlockSpec((1, NPAD, D), lambda i: (i, 0, 0)),
        out_shape=jax.ShapeDtypeStruct((B, NPAD, D), jnp.float32),
    )(p, We.astype(jnp.bfloat16), be.reshape(1, D),
      Wg[:D].astype(jnp.bfloat16), Wg[D:].astype(jnp.bfloat16),
      bg.reshape(1, D))

    amp_low = out[:, :N, :]
    return (amp_src, amp_low)
